# baseline (speedup 1.0000x reference)
import functools

import jax
import jax.numpy as jnp
from jax.experimental import pallas as pl
from jax.experimental.pallas import tpu as pltpu

_C = 32
_TILE = 18432
_LN = _TILE // 128


def _part_kernel(net_ref, tgt_ref, stu_ref, tea_ref, sums_ref, cnts_ref):
    t = pl.program_id(1)
    n0 = net_ref[0, 0:1, :]
    n1 = net_ref[0, 1:2, :]
    n2 = net_ref[0, 2:3, :]
    tgt = tgt_ref[0, 0:1, :]
    pred0 = (n0 >= n1) & (n0 >= n2)
    pred1 = (~pred0) & (n1 >= n2)
    kid = (tgt == 1) & pred0
    is2 = tgt == 2
    tum = is2 & pred1
    wrong = is2 & (~pred1)
    kidf = kid.astype(jnp.float32).reshape(1, _LN, 128)
    tumf = tum.astype(jnp.float32).reshape(1, _LN, 128)
    wrongf = wrong.astype(jnp.float32).reshape(1, _LN, 128)
    stu = stu_ref[0].reshape(_C, _LN, 128)
    tea = tea_ref[0].reshape(_C, _LN, 128)
    kid_part = jnp.sum(stu * kidf, axis=1)
    wrong_part = jnp.sum(stu * wrongf, axis=1)
    tum_part = jnp.sum(tea * tumf, axis=1)
    sums = jnp.stack([kid_part, tum_part, wrong_part], axis=0)
    cnts = jnp.concatenate(
        [jnp.sum(kidf, axis=1), jnp.sum(tumf, axis=1), jnp.sum(wrongf, axis=1)],
        axis=0,
    )

    @pl.when(t == 0)
    def _():
        sums_ref[0] = sums
        cnts_ref[0] = cnts

    @pl.when(t != 0)
    def _():
        sums_ref[0] += sums
        cnts_ref[0] += cnts


def _norm(v):
    return v / (jnp.sqrt(jnp.sum(v * v, axis=-1, keepdims=True)) + 1e-8)


def _loss_kernel(spatial, nb, nd, sums_ref, cnts_ref, deque_ref, out_ref):
    sums = sums_ref[:]
    cnts = jnp.sum(cnts_ref[:], axis=-1)
    vecs = jnp.sum(sums, axis=-1) / spatial
    kid_n = _norm(vecs[:, 0, :])
    tum_n = _norm(vecs[:, 1, :])
    tgt_n = _norm(vecs[:, 2, :])
    dq_n = _norm(deque_ref[:])
    ext = jnp.concatenate([dq_n, kid_n], axis=0)
    kid_sim = jax.lax.dot_general(
        tgt_n, ext, (((1,), (1,)), ((), ())),
        preferred_element_type=jnp.float32)
    tum_sim = jnp.sum(tgt_n * tum_n, axis=-1, keepdims=True)
    active_f = ((cnts[:, 1:2] != 0).astype(jnp.float32)
                * (cnts[:, 2:3] != 0).astype(jnp.float32))
    iext = jax.lax.broadcasted_iota(jnp.int32, (nb, nd + nb), 0)
    jext = jax.lax.broadcasted_iota(jnp.int32, (nb, nd + nb), 1)
    valid_f = ((jext - nd) <= iext).astype(jnp.float32)
    for j in range(nb):
        kvf = jnp.where(cnts[j, 0] != 0, 1.0, 0.0)
        valid_f = valid_f * jnp.where(jext == nd + j, kvf, 1.0)
    exp_t = active_f * jnp.exp(tum_sim)
    exp_k = active_f * valid_f * jnp.exp(kid_sim)
    check = jnp.sum(active_f) > 0.0
    loss = jnp.where(
        check,
        (-1.0 / nb) * jnp.log(jnp.sum(exp_t) / jnp.sum(exp_k)),
        0.0,
    )
    out_ref[:, :] = jnp.full((1, 1), loss, jnp.float32)


def kernel(net_output, student_feature, teacher_feature, target, kidney_deque):
    B = net_output.shape[0]
    spatial = net_output.shape[2] * net_output.shape[3] * net_output.shape[4]
    D = kidney_deque.shape[0]
    net = net_output.reshape(B, 3, spatial)
    tgt = target.reshape(B, 1, spatial)
    stu = student_feature.reshape(B, _C, spatial)
    tea = teacher_feature.reshape(B, _C, spatial)
    nt = spatial // _TILE
    sums, cnts = pl.pallas_call(
        _part_kernel,
        grid=(B, nt),
        in_specs=[
            pl.BlockSpec((1, 3, _TILE), lambda b, t: (b, 0, t)),
            pl.BlockSpec((1, 1, _TILE), lambda b, t: (b, 0, t)),
            pl.BlockSpec((1, _C, _TILE), lambda b, t: (b, 0, t)),
            pl.BlockSpec((1, _C, _TILE), lambda b, t: (b, 0, t)),
        ],
        out_specs=[
            pl.BlockSpec((1, 3, _C, 128), lambda b, t: (b, 0, 0, 0)),
            pl.BlockSpec((1, 3, 128), lambda b, t: (b, 0, 0)),
        ],
        out_shape=[
            jax.ShapeDtypeStruct((B, 3, _C, 128), jnp.float32),
            jax.ShapeDtypeStruct((B, 3, 128), jnp.float32),
        ],
        compiler_params=pltpu.CompilerParams(
            dimension_semantics=("parallel", "arbitrary")),
    )(net, tgt, stu, tea)
    loss = pl.pallas_call(
        functools.partial(_loss_kernel, float(spatial), B, D),
        out_shape=jax.ShapeDtypeStruct((1, 1), jnp.float32),
    )(sums, cnts, kidney_deque)
    return loss[0, 0]


# speedup vs baseline: 1.1238x; 1.1238x over previous
import functools

import jax
import jax.numpy as jnp
from jax.experimental import pallas as pl
from jax.experimental.pallas import tpu as pltpu

_C = 32
_GT = 18
_NG = 96 ** 3 // 1024


def _part_kernel(net_ref, tgt_ref, stu_ref, tea_ref, sums_ref, cnts_ref):
    t = pl.program_id(1)
    n0 = net_ref[0, 0:1]
    n1 = net_ref[0, 1:2]
    n2 = net_ref[0, 2:3]
    tgt = tgt_ref[0, 0:1]
    pred0 = (n0 >= n1) & (n0 >= n2)
    pred1 = (~pred0) & (n1 >= n2)
    kid = (tgt == 1) & pred0
    is2 = tgt == 2
    tum = is2 & pred1
    wrong = is2 & (~pred1)
    kidf = kid.astype(jnp.float32)
    tumf = tum.astype(jnp.float32)
    wrongf = wrong.astype(jnp.float32)
    stu = stu_ref[0]
    tea = tea_ref[0]
    kid_part = jnp.sum(stu * kidf, axis=1)
    wrong_part = jnp.sum(stu * wrongf, axis=1)
    tum_part = jnp.sum(tea * tumf, axis=1)
    sums = jnp.stack([kid_part, tum_part, wrong_part], axis=0)
    cnts = jnp.concatenate(
        [jnp.sum(kidf, axis=1), jnp.sum(tumf, axis=1), jnp.sum(wrongf, axis=1)],
        axis=0,
    )

    @pl.when(t == 0)
    def _():
        sums_ref[0] = sums
        cnts_ref[0] = cnts

    @pl.when(t != 0)
    def _():
        sums_ref[0] += sums
        cnts_ref[0] += cnts


def _norm(v):
    return v / (jnp.sqrt(jnp.sum(v * v, axis=-1, keepdims=True)) + 1e-8)


def _loss_kernel(spatial, nb, nd, sums_ref, cnts_ref, deque_ref, out_ref):
    sums = sums_ref[:]
    cnts = jnp.sum(cnts_ref[:], axis=(-2, -1))
    vecs = jnp.sum(sums, axis=(-2, -1)) / spatial
    kid_n = _norm(vecs[:, 0, :])
    tum_n = _norm(vecs[:, 1, :])
    tgt_n = _norm(vecs[:, 2, :])
    dq_n = _norm(deque_ref[:])
    ext = jnp.concatenate([dq_n, kid_n], axis=0)
    kid_sim = jax.lax.dot_general(
        tgt_n, ext, (((1,), (1,)), ((), ())),
        preferred_element_type=jnp.float32)
    tum_sim = jnp.sum(tgt_n * tum_n, axis=-1, keepdims=True)
    active_f = ((cnts[:, 1:2] != 0).astype(jnp.float32)
                * (cnts[:, 2:3] != 0).astype(jnp.float32))
    iext = jax.lax.broadcasted_iota(jnp.int32, (nb, nd + nb), 0)
    jext = jax.lax.broadcasted_iota(jnp.int32, (nb, nd + nb), 1)
    valid_f = ((jext - nd) <= iext).astype(jnp.float32)
    for j in range(nb):
        kvf = jnp.where(cnts[j, 0] != 0, 1.0, 0.0)
        valid_f = valid_f * jnp.where(jext == nd + j, kvf, 1.0)
    exp_t = active_f * jnp.exp(tum_sim)
    exp_k = active_f * valid_f * jnp.exp(kid_sim)
    check = jnp.sum(active_f) > 0.0
    loss = jnp.where(
        check,
        (-1.0 / nb) * jnp.log(jnp.sum(exp_t) / jnp.sum(exp_k)),
        0.0,
    )
    out_ref[:, :] = jnp.full((1, 1), loss, jnp.float32)


def kernel(net_output, student_feature, teacher_feature, target, kidney_deque):
    B = net_output.shape[0]
    spatial = net_output.shape[2] * net_output.shape[3] * net_output.shape[4]
    ng = spatial // 1024
    D = kidney_deque.shape[0]
    net = net_output.reshape(B, 3, ng, 8, 128)
    tgt = target.reshape(B, 1, ng, 8, 128)
    stu = student_feature.reshape(B, _C, ng, 8, 128)
    tea = teacher_feature.reshape(B, _C, ng, 8, 128)
    nt = ng // _GT
    sums, cnts = pl.pallas_call(
        _part_kernel,
        grid=(B, nt),
        in_specs=[
            pl.BlockSpec((1, 3, _GT, 8, 128), lambda b, t: (b, 0, t, 0, 0)),
            pl.BlockSpec((1, 1, _GT, 8, 128), lambda b, t: (b, 0, t, 0, 0)),
            pl.BlockSpec((1, _C, _GT, 8, 128), lambda b, t: (b, 0, t, 0, 0)),
            pl.BlockSpec((1, _C, _GT, 8, 128), lambda b, t: (b, 0, t, 0, 0)),
        ],
        out_specs=[
            pl.BlockSpec((1, 3, _C, 8, 128), lambda b, t: (b, 0, 0, 0, 0)),
            pl.BlockSpec((1, 3, 8, 128), lambda b, t: (b, 0, 0, 0)),
        ],
        out_shape=[
            jax.ShapeDtypeStruct((B, 3, _C, 8, 128), jnp.float32),
            jax.ShapeDtypeStruct((B, 3, 8, 128), jnp.float32),
        ],
        compiler_params=pltpu.CompilerParams(
            dimension_semantics=("parallel", "arbitrary")),
    )(net, tgt, stu, tea)
    loss = pl.pallas_call(
        functools.partial(_loss_kernel, float(spatial), B, D),
        out_shape=jax.ShapeDtypeStruct((1, 1), jnp.float32),
    )(sums, cnts, kidney_deque)
    return loss[0, 0]


# speedup vs baseline: 4.8478x; 4.3139x over previous
import functools

import jax
import jax.numpy as jnp
from jax.experimental import pallas as pl
from jax.experimental.pallas import tpu as pltpu

_C = 32
_DZ = 4


def _part_kernel(net_ref, tgt_ref, stu_ref, tea_ref, sums_ref, cnts_ref):
    t = pl.program_id(1)
    n0 = net_ref[0, 0:1]
    n1 = net_ref[0, 1:2]
    n2 = net_ref[0, 2:3]
    tgt = tgt_ref[0, 0:1]
    pred0 = (n0 >= n1) & (n0 >= n2)
    pred1 = (~pred0) & (n1 >= n2)
    kid = (tgt == 1) & pred0
    is2 = tgt == 2
    tum = is2 & pred1
    wrong = is2 & (~pred1)
    kidf = kid.astype(jnp.float32).reshape(1, _DZ, 12, 8, 96)
    tumf = tum.astype(jnp.float32).reshape(1, _DZ, 12, 8, 96)
    wrongf = wrong.astype(jnp.float32).reshape(1, _DZ, 12, 8, 96)
    stu = stu_ref[0].reshape(_C, _DZ, 12, 8, 96)
    tea = tea_ref[0].reshape(_C, _DZ, 12, 8, 96)
    kid_part = jnp.sum(stu * kidf, axis=(1, 2))
    wrong_part = jnp.sum(stu * wrongf, axis=(1, 2))
    tum_part = jnp.sum(tea * tumf, axis=(1, 2))
    sums = jnp.stack([kid_part, tum_part, wrong_part], axis=0)
    cnts = jnp.concatenate(
        [jnp.sum(kidf, axis=(1, 2)), jnp.sum(tumf, axis=(1, 2)),
         jnp.sum(wrongf, axis=(1, 2))],
        axis=0,
    )

    @pl.when(t == 0)
    def _():
        sums_ref[0] = sums
        cnts_ref[0] = cnts

    @pl.when(t != 0)
    def _():
        sums_ref[0] += sums
        cnts_ref[0] += cnts


def _norm(v):
    return v / (jnp.sqrt(jnp.sum(v * v, axis=-1, keepdims=True)) + 1e-8)


def _loss_kernel(spatial, nb, nd, sums_ref, cnts_ref, deque_ref, out_ref):
    sums = sums_ref[:]
    cnts = jnp.sum(cnts_ref[:], axis=(-2, -1))
    vecs = jnp.sum(sums, axis=(-2, -1)) / spatial
    kid_n = _norm(vecs[:, 0, :])
    tum_n = _norm(vecs[:, 1, :])
    tgt_n = _norm(vecs[:, 2, :])
    dq_n = _norm(deque_ref[:])
    ext = jnp.concatenate([dq_n, kid_n], axis=0)
    kid_sim = jax.lax.dot_general(
        tgt_n, ext, (((1,), (1,)), ((), ())),
        preferred_element_type=jnp.float32)
    tum_sim = jnp.sum(tgt_n * tum_n, axis=-1, keepdims=True)
    active_f = ((cnts[:, 1:2] != 0).astype(jnp.float32)
                * (cnts[:, 2:3] != 0).astype(jnp.float32))
    iext = jax.lax.broadcasted_iota(jnp.int32, (nb, nd + nb), 0)
    jext = jax.lax.broadcasted_iota(jnp.int32, (nb, nd + nb), 1)
    valid_f = ((jext - nd) <= iext).astype(jnp.float32)
    for j in range(nb):
        kvf = jnp.where(cnts[j, 0] != 0, 1.0, 0.0)
        valid_f = valid_f * jnp.where(jext == nd + j, kvf, 1.0)
    exp_t = active_f * jnp.exp(tum_sim)
    exp_k = active_f * valid_f * jnp.exp(kid_sim)
    check = jnp.sum(active_f) > 0.0
    loss = jnp.where(
        check,
        (-1.0 / nb) * jnp.log(jnp.sum(exp_t) / jnp.sum(exp_k)),
        0.0,
    )
    out_ref[:, :] = jnp.full((1, 1), loss, jnp.float32)


def kernel(net_output, student_feature, teacher_feature, target, kidney_deque):
    B = net_output.shape[0]
    nz = net_output.shape[2]
    spatial = net_output.shape[2] * net_output.shape[3] * net_output.shape[4]
    D = kidney_deque.shape[0]
    nt = nz // _DZ
    sums, cnts = pl.pallas_call(
        _part_kernel,
        grid=(B, nt),
        in_specs=[
            pl.BlockSpec((1, 3, _DZ, 96, 96), lambda b, t: (b, 0, t, 0, 0)),
            pl.BlockSpec((1, 1, _DZ, 96, 96), lambda b, t: (b, 0, t, 0, 0)),
            pl.BlockSpec((1, _C, _DZ, 96, 96), lambda b, t: (b, 0, t, 0, 0)),
            pl.BlockSpec((1, _C, _DZ, 96, 96), lambda b, t: (b, 0, t, 0, 0)),
        ],
        out_specs=[
            pl.BlockSpec((1, 3, _C, 8, 96), lambda b, t: (b, 0, 0, 0, 0)),
            pl.BlockSpec((1, 3, 8, 96), lambda b, t: (b, 0, 0, 0)),
        ],
        out_shape=[
            jax.ShapeDtypeStruct((B, 3, _C, 8, 96), jnp.float32),
            jax.ShapeDtypeStruct((B, 3, 8, 96), jnp.float32),
        ],
        compiler_params=pltpu.CompilerParams(
            dimension_semantics=("parallel", "arbitrary")),
    )(net_output, target, student_feature, teacher_feature)
    loss = pl.pallas_call(
        functools.partial(_loss_kernel, float(spatial), B, D),
        out_shape=jax.ShapeDtypeStruct((1, 1), jnp.float32),
    )(sums, cnts, kidney_deque)
    return loss[0, 0]
